# baseline (speedup 1.0000x reference)
import jax
import jax.numpy as jnp
from jax.experimental import pallas as pl
from jax.experimental.pallas import tpu as pltpu

_MAX_K = 128
_SEQ = 2048
_D = 64
_T_ROWS = 2 * _MAX_K + 1
_ROWS128 = _SEQ * _D // 128
_HALF = _SEQ // 2


def _band_body(w_ref, out_ref, img_ref, f_ref, sem_v, sem_h, sem_i):
    w = w_ref[...]
    c00 = jnp.concatenate([w[0:1, :], w[0:1, :]], axis=1)
    czz = jnp.concatenate([w[_T_ROWS - 1:, :], w[_T_ROWS - 1:, :]], axis=1)
    k_i = jax.lax.broadcasted_iota(jnp.int32, (128, 256), 0)
    r_i = jax.lax.broadcasted_iota(jnp.int32, (128, 256), 1)
    p_even = (r_i == 2 * k_i).astype(jnp.float32)
    p_odd = (r_i == 2 * k_i + 1).astype(jnp.float32)
    dot = lambda p, m: jax.lax.dot_general(
        p, m, (((1,), (0,)), ((), ())), preferred_element_type=jnp.float32)
    w1 = w[1:257, :]
    w0 = w[0:256, :]
    f_ref[0, 0:959, :] = jnp.broadcast_to(c00, (959, 128))
    f_ref[0, 959:1087, :] = jnp.concatenate([dot(p_even, w0), dot(p_odd, w0)],
                                            axis=1)
    f_ref[0, 1087:2048, :] = jnp.broadcast_to(czz, (961, 128))
    f_ref[1, 0:960, :] = jnp.broadcast_to(c00, (960, 128))
    f_ref[1, 960:1088, :] = jnp.concatenate([dot(p_even, w1), dot(p_odd, w1)],
                                            axis=1)
    f_ref[1, 1088:2048, :] = jnp.broadcast_to(czz, (960, 128))

    stage = pltpu.make_async_copy(f_ref.at[:, pl.ds(0, _SEQ), :],
                                  img_ref.at[:, pl.ds(0, _SEQ), :], sem_i)
    stage.start()
    stage.wait()

    def issue(s, _):
        off_t = _ROWS128 - 1 - s
        off_b = _ROWS128 // 2 - 1 - s
        for pl_i, dq in ((1, 1), (0, 0)):
            pltpu.make_async_copy(
                f_ref.at[pl_i, pl.ds(off_t, _ROWS128), :],
                out_ref.at[2 * s + dq], sem_v).start()
            pltpu.make_async_copy(
                img_ref.at[pl_i, pl.ds(off_b, _ROWS128), :],
                out_ref.at[_SEQ // 2 + 2 * s + dq], sem_h).start()
        return 0

    jax.lax.fori_loop(0, _HALF // 2, issue, 0)

    def drain(s, _):
        pltpu.make_async_copy(f_ref.at[0, pl.ds(0, _ROWS128), :],
                              out_ref.at[0], sem_v).wait()
        pltpu.make_async_copy(img_ref.at[0, pl.ds(0, _ROWS128), :],
                              out_ref.at[0], sem_h).wait()
        return 0

    jax.lax.fori_loop(0, _HALF, drain, 0)


def kernel(seq_len, emb_weight):
    del seq_len
    out, _ = pl.pallas_call(
        _band_body,
        grid=(1,),
        in_specs=[pl.BlockSpec((_T_ROWS, _D), lambda i: (0, 0))],
        out_specs=[pl.BlockSpec(memory_space=pltpu.MemorySpace.HBM),
                   pl.BlockSpec(memory_space=pltpu.MemorySpace.HBM)],
        out_shape=[jax.ShapeDtypeStruct((_SEQ, _ROWS128, 128), jnp.float32),
                   jax.ShapeDtypeStruct((2, _SEQ, 128), jnp.float32)],
        scratch_shapes=[pltpu.VMEM((2, _SEQ, 128), jnp.float32),
                        pltpu.SemaphoreType.DMA,
                        pltpu.SemaphoreType.DMA,
                        pltpu.SemaphoreType.DMA],
    )(emb_weight)
    return out.reshape(_SEQ, _SEQ, _D)


# speedup vs baseline: 5.2718x; 5.2718x over previous
import functools

import jax
import jax.numpy as jnp
from jax import lax
from jax.experimental import pallas as pl
from jax.experimental.pallas import tpu as pltpu
from jax.experimental.pallas import tpu_sc as plsc

_MAX_K = 128
_SEQ = 2048
_D = 64
_T_ROWS = 2 * _MAX_K + 1
_ROWS128 = _SEQ * _D // 128
_SPLIT = 1024


def _tc_body(w_ref, out_ref, f_ref, sem):
    w = w_ref[...]
    c00 = jnp.concatenate([w[0:1, :], w[0:1, :]], axis=1)
    czz = jnp.concatenate([w[_T_ROWS - 1:, :], w[_T_ROWS - 1:, :]], axis=1)
    k_i = jax.lax.broadcasted_iota(jnp.int32, (128, 256), 0)
    r_i = jax.lax.broadcasted_iota(jnp.int32, (128, 256), 1)
    p_even = (r_i == 2 * k_i).astype(jnp.float32)
    p_odd = (r_i == 2 * k_i + 1).astype(jnp.float32)
    dot = lambda p, m: jax.lax.dot_general(
        p, m, (((1,), (0,)), ((), ())), preferred_element_type=jnp.float32)
    w1 = w[1:257, :]
    w0 = w[0:256, :]
    f_ref[0, 0:959, :] = jnp.broadcast_to(c00, (959, 128))
    f_ref[0, 959:1087, :] = jnp.concatenate([dot(p_even, w0), dot(p_odd, w0)],
                                            axis=1)
    f_ref[0, 1087:2048, :] = jnp.broadcast_to(czz, (961, 128))
    f_ref[1, 0:960, :] = jnp.broadcast_to(c00, (960, 128))
    f_ref[1, 960:1088, :] = jnp.concatenate([dot(p_even, w1), dot(p_odd, w1)],
                                            axis=1)
    f_ref[1, 1088:2048, :] = jnp.broadcast_to(czz, (960, 128))

    def issue(s, _):
        off = _ROWS128 - 1 - s
        pltpu.make_async_copy(f_ref.at[1, pl.ds(off, _ROWS128), :],
                              out_ref.at[2 * s + 1], sem).start()
        pltpu.make_async_copy(f_ref.at[0, pl.ds(off, _ROWS128), :],
                              out_ref.at[2 * s], sem).start()
        return 0

    jax.lax.fori_loop(0, _SPLIT // 2, issue, 0)

    def drain(s, _):
        pltpu.make_async_copy(f_ref.at[0, pl.ds(0, _ROWS128), :],
                              out_ref.at[0], sem).wait()
        return 0

    jax.lax.fori_loop(0, _SPLIT, drain, 0)


def _tc_half(emb_weight):
    out = pl.pallas_call(
        _tc_body,
        grid=(1,),
        in_specs=[pl.BlockSpec((_T_ROWS, _D), lambda i: (0, 0))],
        out_specs=pl.BlockSpec(memory_space=pltpu.MemorySpace.HBM),
        out_shape=jax.ShapeDtypeStruct((_SPLIT, _ROWS128, 128), jnp.float32),
        scratch_shapes=[pltpu.VMEM((2, _SEQ, 128), jnp.float32),
                        pltpu.SemaphoreType.DMA],
    )(emb_weight)
    return out.reshape(_SPLIT, _SEQ, _D)


_NQ_SC = _SEQ - _SPLIT
_Q_PER_TILE = _NQ_SC // 32
_HALF_W = (_SEQ // 2) * _D
_WIN = 1024 + _Q_PER_TILE
_MARG = _T_ROWS
_EXT = _WIN + 2 * _MARG
_RING = 4


def _sc_body(w_hbm, out_hbm, wbuf, wext, sem):
    c = lax.axis_index("c")
    s = lax.axis_index("s")
    wid = s * 2 + c
    q0 = _SPLIT + wid * _Q_PER_TILE

    pltpu.sync_copy(w_hbm.at[pl.ds(0, _D)], wbuf.at[pl.ds(0, _D)])
    pltpu.sync_copy(w_hbm.at[pl.ds(256 * _D, _D)], wbuf.at[pl.ds(_D, _D)])
    c0 = [wbuf[pl.ds(j * 16, 16)] for j in range(4)]
    cz = [wbuf[pl.ds(_D + j * 16, 16)] for j in range(4)]

    def _drain_one():
        pltpu.make_async_copy(wext.at[pl.ds(0, _HALF_W)],
                              out_hbm.at[pl.ds(0, _HALF_W)], sem).wait()

    for h in (0, 1):
        lo = 1024 * h + _SEQ - 1 - (q0 + _Q_PER_TILE - 1)
        p = 1919 - lo
        a = jnp.clip(p, 0, _WIN)
        b = jnp.clip(p + _T_ROWS, 0, _WIN)
        pc = jnp.clip(p, -_MARG, _WIN + _MARG - _T_ROWS)

        def fill(vj):
            def body(i, _):
                for j in range(4):
                    wext[pl.ds((_MARG + i) * _D + j * 16, 16)] = vj[j]
                return 0
            return body

        lax.fori_loop(0, a, fill(c0), 0)
        lax.fori_loop(b, _WIN, fill(cz), 0)
        pltpu.sync_copy(w_hbm,
                        wext.at[pl.ds((_MARG + pc) * _D, _T_ROWS * _D)])

        def _start(k):
            src = wext.at[pl.ds((_MARG + _Q_PER_TILE - 1 - k) * _D, _HALF_W)]
            dst = out_hbm.at[pl.ds((2 * (q0 - _SPLIT + k) + h) * _HALF_W,
                                   _HALF_W)]
            pltpu.async_copy(src, dst, sem)

        for j in range(_RING):
            _start(j)

        def _steady(k, _):
            _drain_one()
            _start(_RING + k)
            return 0

        lax.fori_loop(0, _Q_PER_TILE - _RING, _steady, 0)
        for j in range(_RING):
            _drain_one()


def _sc_half(emb_weight):
    mesh = plsc.VectorSubcoreMesh(core_axis_name="c", subcore_axis_name="s")
    run = functools.partial(
        pl.kernel,
        mesh=mesh,
        out_type=jax.ShapeDtypeStruct((2 * _NQ_SC * _HALF_W,), jnp.float32),
        scratch_types=[
            pltpu.VMEM((2 * _D,), jnp.float32),
            pltpu.VMEM((_EXT * _D,), jnp.float32),
            pltpu.SemaphoreType.DMA,
        ],
    )(_sc_body)
    out = run(emb_weight.reshape(-1))
    return out.reshape(_NQ_SC, _SEQ, _D)


def kernel(seq_len, emb_weight):
    del seq_len
    top = _tc_half(emb_weight)
    bot = _sc_half(emb_weight)
    return jnp.concatenate([top, bot], axis=0)


# speedup vs baseline: 6.7248x; 1.2756x over previous
import functools

import jax
import jax.numpy as jnp
from jax import lax
from jax.experimental import pallas as pl
from jax.experimental.pallas import tpu as pltpu
from jax.experimental.pallas import tpu_sc as plsc

_MAX_K = 128
_SEQ = 2048
_D = 64
_T_ROWS = 2 * _MAX_K + 1
_ROWS128 = _SEQ * _D // 128
_SPLIT = 0


def _tc_body(w_ref, out_ref, f_ref, sem):
    w = w_ref[...]
    c00 = jnp.concatenate([w[0:1, :], w[0:1, :]], axis=1)
    czz = jnp.concatenate([w[_T_ROWS - 1:, :], w[_T_ROWS - 1:, :]], axis=1)
    k_i = jax.lax.broadcasted_iota(jnp.int32, (128, 256), 0)
    r_i = jax.lax.broadcasted_iota(jnp.int32, (128, 256), 1)
    p_even = (r_i == 2 * k_i).astype(jnp.float32)
    p_odd = (r_i == 2 * k_i + 1).astype(jnp.float32)
    dot = lambda p, m: jax.lax.dot_general(
        p, m, (((1,), (0,)), ((), ())), preferred_element_type=jnp.float32)
    w1 = w[1:257, :]
    w0 = w[0:256, :]
    f_ref[0, 0:959, :] = jnp.broadcast_to(c00, (959, 128))
    f_ref[0, 959:1087, :] = jnp.concatenate([dot(p_even, w0), dot(p_odd, w0)],
                                            axis=1)
    f_ref[0, 1087:2048, :] = jnp.broadcast_to(czz, (961, 128))
    f_ref[1, 0:960, :] = jnp.broadcast_to(c00, (960, 128))
    f_ref[1, 960:1088, :] = jnp.concatenate([dot(p_even, w1), dot(p_odd, w1)],
                                            axis=1)
    f_ref[1, 1088:2048, :] = jnp.broadcast_to(czz, (960, 128))

    def issue(s, _):
        off = _ROWS128 - 1 - s
        pltpu.make_async_copy(f_ref.at[1, pl.ds(off, _ROWS128), :],
                              out_ref.at[2 * s + 1], sem).start()
        pltpu.make_async_copy(f_ref.at[0, pl.ds(off, _ROWS128), :],
                              out_ref.at[2 * s], sem).start()
        return 0

    jax.lax.fori_loop(0, _SPLIT // 2, issue, 0)

    def drain(s, _):
        pltpu.make_async_copy(f_ref.at[0, pl.ds(0, _ROWS128), :],
                              out_ref.at[0], sem).wait()
        return 0

    jax.lax.fori_loop(0, _SPLIT, drain, 0)


def _tc_half(emb_weight):
    out = pl.pallas_call(
        _tc_body,
        grid=(1,),
        in_specs=[pl.BlockSpec((_T_ROWS, _D), lambda i: (0, 0))],
        out_specs=pl.BlockSpec(memory_space=pltpu.MemorySpace.HBM),
        out_shape=jax.ShapeDtypeStruct((_SPLIT, _ROWS128, 128), jnp.float32),
        scratch_shapes=[pltpu.VMEM((2, _SEQ, 128), jnp.float32),
                        pltpu.SemaphoreType.DMA],
    )(emb_weight)
    return out.reshape(_SPLIT, _SEQ, _D)


_NQ_SC = _SEQ - _SPLIT
_Q_PER_TILE = _NQ_SC // 32
_HALF_W = (_SEQ // 2) * _D
_WIN = 1024 + _Q_PER_TILE
_MARG = _T_ROWS
_EXT = _WIN + 2 * _MARG
_RING = 8


def _sc_body(w_hbm, out_hbm, wbuf, wext, sem):
    c = lax.axis_index("c")
    s = lax.axis_index("s")
    wid = s * 2 + c
    q0 = _SPLIT + wid * _Q_PER_TILE

    pltpu.sync_copy(w_hbm.at[pl.ds(0, _D)], wbuf.at[pl.ds(0, _D)])
    pltpu.sync_copy(w_hbm.at[pl.ds(256 * _D, _D)], wbuf.at[pl.ds(_D, _D)])
    c0 = [wbuf[pl.ds(j * 16, 16)] for j in range(4)]
    cz = [wbuf[pl.ds(_D + j * 16, 16)] for j in range(4)]

    def _drain_one():
        pltpu.make_async_copy(wext.at[pl.ds(0, _HALF_W)],
                              out_hbm.at[pl.ds(0, _HALF_W)], sem).wait()

    for h in (0, 1):
        lo = 1024 * h + _SEQ - 1 - (q0 + _Q_PER_TILE - 1)
        p = 1919 - lo
        a = jnp.clip(p, 0, _WIN)
        b = jnp.clip(p + _T_ROWS, 0, _WIN)
        pc = jnp.clip(p, -_MARG, _WIN + _MARG - _T_ROWS)

        def fill(vj):
            def body(i, _):
                for j in range(4):
                    wext[pl.ds((_MARG + i) * _D + j * 16, 16)] = vj[j]
                return 0
            return body

        lax.fori_loop(0, a, fill(c0), 0)
        lax.fori_loop(b, _WIN, fill(cz), 0)
        pltpu.sync_copy(w_hbm,
                        wext.at[pl.ds((_MARG + pc) * _D, _T_ROWS * _D)])

        def _start(k):
            src = wext.at[pl.ds((_MARG + _Q_PER_TILE - 1 - k) * _D, _HALF_W)]
            dst = out_hbm.at[pl.ds((2 * (q0 - _SPLIT + k) + h) * _HALF_W,
                                   _HALF_W)]
            pltpu.async_copy(src, dst, sem)

        for j in range(_RING):
            _start(j)

        def _steady(k, _):
            _drain_one()
            _start(_RING + k)
            return 0

        lax.fori_loop(0, _Q_PER_TILE - _RING, _steady, 0)
        for j in range(_RING):
            _drain_one()


def _sc_half(emb_weight):
    mesh = plsc.VectorSubcoreMesh(core_axis_name="c", subcore_axis_name="s")
    run = functools.partial(
        pl.kernel,
        mesh=mesh,
        out_type=jax.ShapeDtypeStruct((2 * _NQ_SC * _HALF_W,), jnp.float32),
        scratch_types=[
            pltpu.VMEM((2 * _D,), jnp.float32),
            pltpu.VMEM((_EXT * _D,), jnp.float32),
            pltpu.SemaphoreType.DMA,
        ],
    )(_sc_body)
    out = run(emb_weight.reshape(-1))
    return out.reshape(_NQ_SC, _SEQ, _D)


def kernel(seq_len, emb_weight):
    del seq_len
    if _SPLIT == 0:
        return _sc_half(emb_weight)
    top = _tc_half(emb_weight)
    bot = _sc_half(emb_weight)
    return jnp.concatenate([top, bot], axis=0)


# speedup vs baseline: 8.7015x; 1.2939x over previous
import jax
import jax.numpy as jnp
from jax.experimental import pallas as pl
from jax.experimental.pallas import tpu as pltpu

_MAX_K = 128
_SEQ = 2048
_D = 64
_T_ROWS = 2 * _MAX_K + 1
_G_ROWS = 2 * _SEQ - 1
_PAD = _SEQ - 1 - _MAX_K


def _band_body(w_ref, out_ref, g_ref, sem):
    g_ref[0:_PAD, :] = jnp.broadcast_to(w_ref[0:1, :], (_PAD, _D))
    g_ref[pl.ds(_PAD, _T_ROWS), :] = w_ref[...]
    g_ref[pl.ds(_PAD + _T_ROWS, _PAD), :] = jnp.broadcast_to(
        w_ref[_T_ROWS - 1:_T_ROWS, :], (_PAD, _D))

    def issue(q, _):
        pltpu.make_async_copy(g_ref.at[pl.ds(_SEQ - 1 - q, _SEQ), :],
                              out_ref.at[q], sem).start()
        return 0

    jax.lax.fori_loop(0, _SEQ, issue, 0)

    def drain(q, _):
        pltpu.make_async_copy(g_ref.at[pl.ds(0, _SEQ), :],
                              out_ref.at[0], sem).wait()
        return 0

    jax.lax.fori_loop(0, _SEQ, drain, 0)


def kernel(seq_len, emb_weight):
    del seq_len
    return pl.pallas_call(
        _band_body,
        grid=(1,),
        in_specs=[pl.BlockSpec((_T_ROWS, _D), lambda i: (0, 0))],
        out_specs=pl.BlockSpec(memory_space=pltpu.MemorySpace.HBM),
        out_shape=jax.ShapeDtypeStruct((_SEQ, _SEQ, _D), jnp.float32),
        scratch_shapes=[pltpu.VMEM((_G_ROWS, _D), jnp.float32),
                        pltpu.SemaphoreType.DMA],
    )(emb_weight)
